# baseline (speedup 1.0000x reference)
import functools

import jax
import jax.numpy as jnp
from jax.experimental import pallas as pl
from jax.experimental.pallas import tpu as pltpu

_G = 8


def _gather_mask_body(ti_ref, tj_ref, *refs):
    g = _G
    a01i = jnp.concatenate([refs[k][0] for k in range(0, g)], axis=0)
    a01j = jnp.concatenate([refs[k][0] for k in range(g, 2 * g)], axis=0)
    a1i = jnp.concatenate([refs[k][0] for k in range(2 * g, 3 * g)], axis=0)
    a1j = jnp.concatenate([refs[k][0] for k in range(3 * g, 4 * g)], axis=0)
    a012i = jnp.concatenate([refs[k][0] for k in range(4 * g, 5 * g)], axis=0)
    a012j = jnp.concatenate([refs[k][0] for k in range(5 * g, 6 * g)], axis=0)
    xi = jnp.concatenate([refs[k][0] for k in range(6 * g, 7 * g)], axis=0)
    xj = jnp.concatenate([refs[k][0] for k in range(7 * g, 8 * g)], axis=0)
    m0_ref, m1_ref, m2_ref, xij_ref = refs[8 * g:8 * g + 4]
    c01 = a01i & a01j
    c1 = a1i & a1j
    c012 = a012i & a012j
    m0_ref[0] = c01 & (c1 ^ 1)
    m1_ref[0] = c1
    m2_ref[0] = c012 & (c01 ^ 1)
    xij_ref[0] = xi * xj


def _spmm_body(m0_ref, m1_ref, m2_ref, xij_ref, x_ref, wt_ref, b_ref, out_ref):
    d = x_ref.shape[1]
    acc = jnp.dot(xij_ref[0], wt_ref[0:d, :], preferred_element_type=jnp.float32)
    for k, mref in enumerate((m0_ref, m1_ref, m2_ref)):
        mk = mref[0].astype(jnp.float32)
        t = jnp.dot(mk, x_ref[...], preferred_element_type=jnp.float32)
        acc = acc + jnp.dot(t, wt_ref[(k + 1) * d:(k + 2) * d, :],
                            preferred_element_type=jnp.float32)
    out_ref[0] = acc + b_ref[0]


@jax.jit
def kernel(x, adj_0_1, adj_1, adj_0_1_2, tar_ei, W, b):
    n, d = x.shape
    bsz = tar_ei.shape[1]
    out_dim = W.shape[0]
    ti = tar_ei[0].astype(jnp.int32)
    tj = tar_ei[1].astype(jnp.int32)

    a01 = adj_0_1.view(jnp.int8).reshape(n, 1, n)
    a1 = adj_1.view(jnp.int8).reshape(n, 1, n)
    a012 = adj_0_1_2.view(jnp.int8).reshape(n, 1, n)
    x3 = x.reshape(n, 1, d)

    def row_spec(width, which, g):
        if which == 0:
            return pl.BlockSpec(
                (1, 1, width), lambda i, ti, tj: (ti[i * _G + g], 0, 0))
        return pl.BlockSpec(
            (1, 1, width), lambda i, ti, tj: (tj[i * _G + g], 0, 0))

    in_specs = []
    operands = []
    for arr in (a01, a1, a012):
        for which in (0, 1):
            for g in range(_G):
                in_specs.append(row_spec(n, which, g))
                operands.append(arr)
    for which in (0, 1):
        for g in range(_G):
            in_specs.append(row_spec(d, which, g))
            operands.append(x3)

    ng = bsz // _G
    out_row = pl.BlockSpec((1, _G, n), lambda i, ti, tj: (i, 0, 0))
    out_xij = pl.BlockSpec((1, _G, d), lambda i, ti, tj: (i, 0, 0))

    m0, m1, m2, xij = pl.pallas_call(
        _gather_mask_body,
        grid_spec=pltpu.PrefetchScalarGridSpec(
            num_scalar_prefetch=2,
            grid=(ng,),
            in_specs=in_specs,
            out_specs=[out_row, out_row, out_row, out_xij],
        ),
        out_shape=[
            jax.ShapeDtypeStruct((ng, _G, n), jnp.int8),
            jax.ShapeDtypeStruct((ng, _G, n), jnp.int8),
            jax.ShapeDtypeStruct((ng, _G, n), jnp.int8),
            jax.ShapeDtypeStruct((ng, _G, d), jnp.float32),
        ],
    )(ti, tj, *operands)

    bg = 128 if bsz % 128 == 0 else bsz
    nb = bsz // bg
    m0r = m0.reshape(nb, bg, n)
    m1r = m1.reshape(nb, bg, n)
    m2r = m2.reshape(nb, bg, n)
    xijr = xij.reshape(nb, bg, d)

    mask_spec = pl.BlockSpec((1, bg, n), lambda i: (i, 0, 0))
    xij_spec = pl.BlockSpec((1, bg, d), lambda i: (i, 0, 0))
    x_spec = pl.BlockSpec((n, d), lambda i: (0, 0))
    wt_spec = pl.BlockSpec((4 * d, out_dim), lambda i: (0, 0))
    b_spec = pl.BlockSpec((1, out_dim), lambda i: (0, 0))

    out = pl.pallas_call(
        _spmm_body,
        grid=(nb,),
        in_specs=[mask_spec, mask_spec, mask_spec, xij_spec, x_spec, wt_spec,
                  b_spec],
        out_specs=pl.BlockSpec((1, bg, out_dim), lambda i: (i, 0, 0)),
        out_shape=jax.ShapeDtypeStruct((nb, bg, out_dim), jnp.float32),
    )(m0r, m1r, m2r, xijr, x, W.T, b.reshape(1, out_dim))

    return out.reshape(bsz, out_dim)
